# baseline (speedup 1.0000x reference)
import math
import functools

import jax
import jax.numpy as jnp
from jax import lax
from jax.experimental import pallas as pl
from jax.experimental.pallas import tpu as pltpu
from jax.experimental.pallas import tpu_sc as plsc

_INFO = plsc.get_sparse_core_info()
_NC = _INFO.num_cores
_NS = _INFO.num_subcores
_NW = _NC * _NS
_LANES = _INFO.num_lanes


def _pe_table(max_length, d):
    position = jnp.arange(max_length, dtype=jnp.float32)[:, None]
    div_term = jnp.exp(
        jnp.arange(0, d, 2, dtype=jnp.float32) * (-math.log(10000.0) / d))
    arg = position * div_term
    pe = jnp.stack([jnp.sin(arg), jnp.cos(arg)], axis=-1)
    return pe.reshape(max_length, d)


def _build_fused_table(token_embedding, pe, scale):
    V, D = token_embedding.shape
    L = pe.shape[0]

    def body(emb_ref, pe_ref, out_ref):
        out_ref[...] = emb_ref[...][0] * scale + pe_ref[...]

    return pl.pallas_call(
        body,
        grid=(V,),
        in_specs=[
            pl.BlockSpec((1, 1, D), lambda t: (t, 0, 0)),
            pl.BlockSpec((L, D), lambda t: (0, 0)),
        ],
        out_specs=pl.BlockSpec((L, D), lambda t: (t, 0)),
        out_shape=jax.ShapeDtypeStruct((V * L, D), jnp.float32),
    )(token_embedding[:, None, :], pe)


def _sc_lookup(input_tokens, fused, B, L):
    R, D = fused.shape
    seq_per_w = B // _NW
    rows_per_w = seq_per_w * L
    chunk = 128
    n_chunks = rows_per_w // chunk
    n_groups = seq_per_w // _LANES

    mesh = plsc.VectorSubcoreMesh(core_axis_name="c", subcore_axis_name="s")

    @functools.partial(
        pl.kernel,
        out_type=jax.ShapeDtypeStruct((B * L, D), jnp.float32),
        mesh=mesh,
        compiler_params=pltpu.CompilerParams(needs_layout_passes=False),
        scratch_types=[
            pltpu.VMEM((seq_per_w * L,), jnp.int32),
            pltpu.VMEM((rows_per_w,), jnp.int32),
            [pltpu.VMEM((chunk, D), jnp.float32) for _ in range(4)],
            [pltpu.SemaphoreType.DMA for _ in range(4)],
            [pltpu.SemaphoreType.DMA for _ in range(4)],
        ],
    )
    def sc_kernel(tok_hbm, fused_hbm, out_hbm, tok_v, idx_v, bufs,
                  gsems, wsems):
        n_buf = len(bufs)
        wid = lax.axis_index("s") * _NC + lax.axis_index("c")
        base_seq = wid * seq_per_w
        pltpu.sync_copy(
            tok_hbm.at[pl.ds(base_seq * L, seq_per_w * L)], tok_v)

        lane = lax.broadcasted_iota(jnp.int32, (_LANES,), 0)
        zeros = jnp.zeros((_LANES,), jnp.int32)

        def scan_group(g):
            base16 = (lane + g * _LANES) * L
            p0 = plsc.load_gather(tok_v, [base16])
            plsc.store_scatter(idx_v, [base16], p0 * L)

            def step(j, carry):
                p, nc, ev = carry
                c = plsc.load_gather(tok_v, [base16 + j])
                nc = jnp.where((c >= 36) & (c <= 41), 2, nc)
                change_lt12 = (p >= 12) | (nc > 0)
                change = jnp.where(c < 12, change_lt12, p < 12)
                nc = jnp.where(c < 12, nc - 1, nc)
                ev = ev + change.astype(jnp.int32)
                plsc.store_scatter(idx_v, [base16 + j], c * L + ev)
                return c, nc, ev

            lax.fori_loop(1, L, step, (p0, zeros, zeros))

        out_base = wid * rows_per_w
        gcopies = [None] * n_chunks
        wcopies = [None] * n_chunks

        def start_gather(k):
            gcopies[k] = pltpu.async_copy(
                fused_hbm.at[idx_v.at[pl.ds(k * chunk, chunk)]],
                bufs[k % n_buf], gsems[k % n_buf])

        def start_write(k):
            gcopies[k].wait()
            wcopies[k] = pltpu.async_copy(
                bufs[k % n_buf],
                out_hbm.at[pl.ds(out_base + k * chunk, chunk)],
                wsems[k % n_buf])

        with jax.named_scope("ev_scan"):
            scan_group(0)
        for k in range(n_buf):
            start_gather(k)
        with jax.named_scope("ev_scan2"):
            for g in range(1, n_groups):
                scan_group(g)

        with jax.named_scope("gather_pipe"):
            for k in range(n_chunks):
                if k >= n_buf:
                    wcopies[k - n_buf].wait()
                    start_gather(k)
                if k >= n_buf - 1:
                    start_write(k - (n_buf - 1))
            for k in range(n_chunks - (n_buf - 1), n_chunks):
                start_write(k)
            for k in range(n_chunks - n_buf, n_chunks):
                wcopies[k].wait()

    return sc_kernel(input_tokens.reshape(B * L), fused)


def kernel(input_tokens, token_embedding):
    B, L = input_tokens.shape
    V, D = token_embedding.shape
    pe = _pe_table(L, D)
    fused = _build_fused_table(token_embedding, pe, math.sqrt(D))
    out = _sc_lookup(input_tokens, fused.reshape(V * L, D), B, L)
    return out.reshape(B, L, D)


# speedup vs baseline: 1.7549x; 1.7549x over previous
import math
import functools

import jax
import jax.numpy as jnp
from jax import lax
from jax.experimental import pallas as pl
from jax.experimental.pallas import tpu as pltpu
from jax.experimental.pallas import tpu_sc as plsc

_INFO = plsc.get_sparse_core_info()
_NC = _INFO.num_cores
_NS = _INFO.num_subcores
_NW = _NC * _NS
_LANES = _INFO.num_lanes


def _pe_table(max_length, d):
    position = jnp.arange(max_length, dtype=jnp.float32)[:, None]
    div_term = jnp.exp(
        jnp.arange(0, d, 2, dtype=jnp.float32) * (-math.log(10000.0) / d))
    arg = position * div_term
    pe = jnp.stack([jnp.sin(arg), jnp.cos(arg)], axis=-1)
    return pe.reshape(max_length, d)


def _build_fused_table(token_embedding, pe, scale):
    V, D = token_embedding.shape
    L = pe.shape[0]

    rows = V * L
    rows_pad = rows + (-rows) % (8 * _NS)

    def body(emb_ref, pe_ref, out_ref):
        pe_block = pe_ref[...]
        for t in range(V):
            out_ref[pl.ds(t * L, L), :] = emb_ref[t] * scale + pe_block
        if rows_pad > rows:
            out_ref[pl.ds(rows, rows_pad - rows), :] = jnp.zeros(
                (rows_pad - rows, D), jnp.float32)

    return pl.pallas_call(
        body,
        out_shape=jax.ShapeDtypeStruct((rows_pad, D), jnp.float32),
    )(token_embedding, pe)


def _sc_lookup(input_tokens, fused, B, L):
    R, D = fused.shape
    seq_per_w = B // _NW
    rows_per_w = seq_per_w * L
    chunk = 128
    n_chunks = rows_per_w // chunk
    n_groups = seq_per_w // _LANES

    mesh = plsc.VectorSubcoreMesh(core_axis_name="c", subcore_axis_name="s")

    @functools.partial(
        pl.kernel,
        out_type=jax.ShapeDtypeStruct((B * L, D), jnp.float32),
        mesh=mesh,
        compiler_params=pltpu.CompilerParams(needs_layout_passes=False),
        scratch_types=[
            pltpu.VMEM((seq_per_w * L,), jnp.int32),
            pltpu.VMEM((rows_per_w,), jnp.int32),
            pltpu.VMEM_SHARED((R, D), jnp.float32),
            [pltpu.VMEM((chunk, D), jnp.float32) for _ in range(2)],
            [pltpu.SemaphoreType.DMA for _ in range(2)],
            [pltpu.SemaphoreType.DMA for _ in range(2)],
            pltpu.SemaphoreType.DMA,
        ],
    )
    def sc_kernel(tok_hbm, fused_hbm, out_hbm, tok_v, idx_v, table_sp, bufs,
                  gsems, wsems, fsem):
        n_buf = len(bufs)
        sid = lax.axis_index("s")
        wid = sid * _NC + lax.axis_index("c")
        base_seq = wid * seq_per_w
        rows_per_sub = R // _NS
        fill = pltpu.async_copy(
            fused_hbm.at[pl.ds(sid * rows_per_sub, rows_per_sub)],
            table_sp.at[pl.ds(sid * rows_per_sub, rows_per_sub)], fsem)
        pltpu.sync_copy(
            tok_hbm.at[pl.ds(base_seq * L, seq_per_w * L)], tok_v)

        lane = lax.broadcasted_iota(jnp.int32, (_LANES,), 0)
        zeros = jnp.zeros((_LANES,), jnp.int32)

        def scan_group(g):
            base16 = (lane + g * _LANES) * L
            p0 = plsc.load_gather(tok_v, [base16])
            plsc.store_scatter(idx_v, [base16], p0 * L)

            def step(j, carry):
                p, nc, ev = carry
                c = plsc.load_gather(tok_v, [base16 + j])
                nc = jnp.where((c >= 36) & (c <= 41), 2, nc)
                change_lt12 = (p >= 12) | (nc > 0)
                change = jnp.where(c < 12, change_lt12, p < 12)
                nc = jnp.where(c < 12, nc - 1, nc)
                ev = ev + change.astype(jnp.int32)
                plsc.store_scatter(idx_v, [base16 + j], c * L + ev)
                return c, nc, ev

            lax.fori_loop(1, L, step, (p0, zeros, zeros))

        out_base = wid * rows_per_w
        gcopies = [None] * n_chunks
        wcopies = [None] * n_chunks

        def start_gather(k):
            gcopies[k] = pltpu.async_copy(
                table_sp.at[idx_v.at[pl.ds(k * chunk, chunk)]],
                bufs[k % n_buf], gsems[k % n_buf])

        def start_write(k):
            gcopies[k].wait()
            wcopies[k] = pltpu.async_copy(
                bufs[k % n_buf],
                out_hbm.at[pl.ds(out_base + k * chunk, chunk)],
                wsems[k % n_buf])

        with jax.named_scope("ev_scan"):
            scan_group(0)
        with jax.named_scope("table_barrier"):
            fill.wait()
            plsc.subcore_barrier()
        for k in range(n_buf):
            start_gather(k)
        with jax.named_scope("ev_scan2"):
            for g in range(1, n_groups):
                scan_group(g)

        with jax.named_scope("gather_pipe"):
            for k in range(n_chunks):
                if k >= n_buf:
                    wcopies[k - n_buf].wait()
                    start_gather(k)
                if k >= n_buf - 1:
                    start_write(k - (n_buf - 1))
            for k in range(n_chunks - (n_buf - 1), n_chunks):
                start_write(k)
            for k in range(n_chunks - n_buf, n_chunks):
                wcopies[k].wait()

    return sc_kernel(input_tokens.reshape(B * L), fused)


def kernel(input_tokens, token_embedding):
    B, L = input_tokens.shape
    V, D = token_embedding.shape
    pe = _pe_table(L, D)
    fused = _build_fused_table(token_embedding, pe, math.sqrt(D))
    out = _sc_lookup(input_tokens, fused, B, L)
    return out.reshape(B, L, D)


# speedup vs baseline: 1.7775x; 1.0128x over previous
import math
import functools

import jax
import jax.numpy as jnp
from jax import lax
from jax.experimental import pallas as pl
from jax.experimental.pallas import tpu as pltpu
from jax.experimental.pallas import tpu_sc as plsc

_INFO = plsc.get_sparse_core_info()
_NC = _INFO.num_cores
_NS = _INFO.num_subcores
_NW = _NC * _NS
_LANES = _INFO.num_lanes


def _pe_table(max_length, d):
    position = jnp.arange(max_length, dtype=jnp.float32)[:, None]
    div_term = jnp.exp(
        jnp.arange(0, d, 2, dtype=jnp.float32) * (-math.log(10000.0) / d))
    arg = position * div_term
    pe = jnp.stack([jnp.sin(arg), jnp.cos(arg)], axis=-1)
    return pe.reshape(max_length, d)


def _build_fused_table(token_embedding, pe, scale):
    V, D = token_embedding.shape
    L = pe.shape[0]

    rows = V * L
    rows_pad = rows + (-rows) % (8 * _NS)

    def body(emb_ref, pe_ref, out_ref):
        pe_block = pe_ref[...]
        for t in range(V):
            out_ref[pl.ds(t * L, L), :] = emb_ref[t] * scale + pe_block
        if rows_pad > rows:
            out_ref[pl.ds(rows, rows_pad - rows), :] = jnp.zeros(
                (rows_pad - rows, D), jnp.float32)

    return pl.pallas_call(
        body,
        out_shape=jax.ShapeDtypeStruct((rows_pad, D), jnp.float32),
    )(token_embedding, pe)


def _sc_lookup(input_tokens, fused, B, L):
    R, D = fused.shape
    seq_per_w = B // _NW
    rows_per_w = seq_per_w * L
    chunk = 128
    n_chunks = rows_per_w // chunk
    n_groups = seq_per_w // _LANES

    mesh = plsc.VectorSubcoreMesh(core_axis_name="c", subcore_axis_name="s")

    @functools.partial(
        pl.kernel,
        out_type=jax.ShapeDtypeStruct((B * L, D), jnp.float32),
        mesh=mesh,
        compiler_params=pltpu.CompilerParams(needs_layout_passes=False),
        scratch_types=[
            pltpu.VMEM((seq_per_w * L,), jnp.int32),
            pltpu.VMEM((rows_per_w,), jnp.int32),
            pltpu.VMEM_SHARED((R, D), jnp.float32),
            [pltpu.VMEM((chunk, D), jnp.float32) for _ in range(2)],
            [pltpu.SemaphoreType.DMA for _ in range(2)],
            [pltpu.SemaphoreType.DMA for _ in range(2)],
            pltpu.SemaphoreType.DMA,
        ],
    )
    def sc_kernel(tok_hbm, fused_hbm, out_hbm, tok_v, idx_v, table_sp, bufs,
                  gsems, wsems, fsem):
        n_buf = len(bufs)
        sid = lax.axis_index("s")
        wid = sid * _NC + lax.axis_index("c")
        base_seq = wid * seq_per_w
        rows_per_sub = R // _NS
        fill = pltpu.async_copy(
            fused_hbm.at[pl.ds(sid * rows_per_sub, rows_per_sub)],
            table_sp.at[pl.ds(sid * rows_per_sub, rows_per_sub)], fsem)
        pltpu.sync_copy(
            tok_hbm.at[pl.ds(base_seq * L, seq_per_w * L)], tok_v)

        lane = lax.broadcasted_iota(jnp.int32, (_LANES,), 0)
        zeros = jnp.zeros((_LANES,), jnp.int32)

        bases = [(lane + g * _LANES) * L for g in range(n_groups)]
        p0s = []
        for base16 in bases:
            p0 = plsc.load_gather(tok_v, [base16])
            plsc.store_scatter(idx_v, [base16], p0 * L)
            p0s.append(p0)

        def step(j, carry):
            new = []
            for g in range(n_groups):
                p, nc, ev = carry[g]
                base16 = bases[g]
                c = plsc.load_gather(tok_v, [base16 + j])
                nc = jnp.where((c >= 36) & (c <= 41), 2, nc)
                change_lt12 = (p >= 12) | (nc > 0)
                change = jnp.where(c < 12, change_lt12, p < 12)
                nc = jnp.where(c < 12, nc - 1, nc)
                ev = ev + change.astype(jnp.int32)
                plsc.store_scatter(idx_v, [base16 + j], c * L + ev)
                new.append((c, nc, ev))
            return tuple(new)

        lax.fori_loop(1, L, step,
                      tuple((p0, zeros, zeros) for p0 in p0s))

        out_base = wid * rows_per_w
        gcopies = [None] * n_chunks
        wcopies = [None] * n_chunks

        def start_gather(k):
            gcopies[k] = pltpu.async_copy(
                table_sp.at[idx_v.at[pl.ds(k * chunk, chunk)]],
                bufs[k % n_buf], gsems[k % n_buf])

        def start_write(k):
            gcopies[k].wait()
            wcopies[k] = pltpu.async_copy(
                bufs[k % n_buf],
                out_hbm.at[pl.ds(out_base + k * chunk, chunk)],
                wsems[k % n_buf])

        fill.wait()
        plsc.subcore_barrier()
        for k in range(n_chunks):
            if k < n_buf:
                start_gather(k)
            else:
                wcopies[k - n_buf].wait()
                start_gather(k)
            if k >= n_buf - 1:
                start_write(k - (n_buf - 1))
        for k in range(n_chunks - (n_buf - 1), n_chunks):
            start_write(k)
        for k in range(n_chunks - n_buf, n_chunks):
            wcopies[k].wait()

    return sc_kernel(input_tokens.reshape(B * L), fused)


def kernel(input_tokens, token_embedding):
    B, L = input_tokens.shape
    V, D = token_embedding.shape
    pe = _pe_table(L, D)
    fused = _build_fused_table(token_embedding, pe, math.sqrt(D))
    out = _sc_lookup(input_tokens, fused, B, L)
    return out.reshape(B, L, D)


# speedup vs baseline: 1.7955x; 1.0101x over previous
import math
import functools

import jax
import jax.numpy as jnp
from jax import lax
from jax.experimental import pallas as pl
from jax.experimental.pallas import tpu as pltpu
from jax.experimental.pallas import tpu_sc as plsc

_INFO = plsc.get_sparse_core_info()
_NC = _INFO.num_cores
_NS = _INFO.num_subcores
_NW = _NC * _NS
_LANES = _INFO.num_lanes


def _pe_table(max_length, d):
    position = jnp.arange(max_length, dtype=jnp.float32)[:, None]
    div_term = jnp.exp(
        jnp.arange(0, d, 2, dtype=jnp.float32) * (-math.log(10000.0) / d))
    arg = position * div_term
    pe = jnp.stack([jnp.sin(arg), jnp.cos(arg)], axis=-1)
    return pe.reshape(max_length, d)


def _build_fused_table(token_embedding, pe, scale):
    V, D = token_embedding.shape
    L = pe.shape[0]

    rows = V * L
    rows_pad = rows + (-rows) % (8 * _NS)

    def body(emb_ref, pe_ref, out_ref):
        pe_block = pe_ref[...]
        for t in range(V):
            out_ref[pl.ds(t * L, L), :] = emb_ref[t] * scale + pe_block
        if rows_pad > rows:
            out_ref[pl.ds(rows, rows_pad - rows), :] = jnp.zeros(
                (rows_pad - rows, D), jnp.float32)

    return pl.pallas_call(
        body,
        out_shape=jax.ShapeDtypeStruct((rows_pad, D), jnp.float32),
    )(token_embedding, pe)


def _sc_lookup(input_tokens, fused, B, L):
    R, D = fused.shape
    seq_per_w = B // _NW
    rows_per_w = seq_per_w * L
    chunk = 128
    n_chunks = rows_per_w // chunk
    n_groups = seq_per_w // _LANES

    mesh = plsc.VectorSubcoreMesh(core_axis_name="c", subcore_axis_name="s")

    @functools.partial(
        pl.kernel,
        out_type=jax.ShapeDtypeStruct((B * L, D), jnp.float32),
        mesh=mesh,
        compiler_params=pltpu.CompilerParams(needs_layout_passes=False),
        scratch_types=[
            pltpu.VMEM((seq_per_w * L,), jnp.int32),
            pltpu.VMEM((n_chunks, chunk), jnp.int32),
            pltpu.VMEM_SHARED((R, D), jnp.float32),
            [pltpu.VMEM((chunk, D), jnp.float32) for _ in range(2)],
            [pltpu.SemaphoreType.DMA for _ in range(2)],
            [pltpu.SemaphoreType.DMA for _ in range(2)],
            pltpu.SemaphoreType.DMA,
        ],
    )
    def sc_kernel(tok_hbm, fused_hbm, out_hbm, tok_v, idx_v, table_sp, bufs,
                  gsems, wsems, fsem):
        n_buf = len(bufs)
        sid = lax.axis_index("s")
        wid = sid * _NC + lax.axis_index("c")
        base_seq = wid * seq_per_w
        rows_per_sub = R // _NS
        fill = pltpu.async_copy(
            fused_hbm.at[pl.ds(sid * rows_per_sub, rows_per_sub)],
            table_sp.at[pl.ds(sid * rows_per_sub, rows_per_sub)], fsem)
        pltpu.sync_copy(
            tok_hbm.at[pl.ds(base_seq * L, seq_per_w * L)], tok_v)

        lane = lax.broadcasted_iota(jnp.int32, (_LANES,), 0)
        zeros = jnp.zeros((_LANES,), jnp.int32)

        bases = [(lane + g * _LANES) * L for g in range(n_groups)]
        p0s = []
        for base16 in bases:
            p0 = plsc.load_gather(tok_v, [base16])
            plsc.store_scatter(idx_v, [base16 >> 7, base16 & 127], p0 * L)
            p0s.append(p0)

        def step(j, carry):
            new = []
            for g in range(n_groups):
                p, nc, ev = carry[g]
                base16 = bases[g]
                c = plsc.load_gather(tok_v, [base16 + j])
                nc = jnp.where((c >= 36) & (c <= 41), 2, nc)
                change_lt12 = (p >= 12) | (nc > 0)
                change = jnp.where(c < 12, change_lt12, p < 12)
                nc = jnp.where(c < 12, nc - 1, nc)
                ev = ev + change.astype(jnp.int32)
                flat = base16 + j
                plsc.store_scatter(idx_v, [flat >> 7, flat & 127], c * L + ev)
                new.append((c, nc, ev))
            return tuple(new)

        lax.fori_loop(1, L, step,
                      tuple((p0, zeros, zeros) for p0 in p0s))

        out_base = wid * rows_per_w

        def start_gather(k, b):
            return pltpu.async_copy(
                table_sp.at[idx_v.at[k]], bufs[b], gsems[b])

        def start_write(k, b):
            return pltpu.async_copy(
                bufs[b], out_hbm.at[pl.ds(out_base + k * chunk, chunk)],
                wsems[b])

        def wait_gather(b):
            pltpu.make_async_copy(
                fused_hbm.at[pl.ds(0, chunk)], bufs[b], gsems[b]).wait()

        def wait_write(b):
            pltpu.make_async_copy(
                fused_hbm.at[pl.ds(0, chunk)], bufs[b], wsems[b]).wait()

        fill.wait()
        plsc.subcore_barrier()

        for b in range(n_buf):
            start_gather(b, b)
        wait_gather(0)
        start_write(0, 0)

        def pipe_body(k, _):
            for b in range(n_buf):

                @pl.when((k % n_buf) == b)
                def _():
                    wait_write(b)
                    start_gather(k, b)

            for b in range(n_buf):

                @pl.when(((k - 1) % n_buf) == b)
                def _():
                    wait_gather(b)
                    start_write(k - 1, b)

            return 0

        lax.fori_loop(n_buf, n_chunks, pipe_body, 0)
        for k in range(n_chunks - 1, n_chunks):
            b = k % n_buf
            wait_gather(b)
            start_write(k, b)
        for k in range(n_chunks - n_buf, n_chunks):
            wait_write(k % n_buf)

    return sc_kernel(input_tokens.reshape(B * L), fused)


def kernel(input_tokens, token_embedding):
    B, L = input_tokens.shape
    V, D = token_embedding.shape
    pe = _pe_table(L, D)
    fused = _build_fused_table(token_embedding, pe, math.sqrt(D))
    out = _sc_lookup(input_tokens, fused, B, L)
    return out.reshape(B, L, D)


# speedup vs baseline: 1.7974x; 1.0011x over previous
import math
import functools

import jax
import jax.numpy as jnp
from jax import lax
from jax.experimental import pallas as pl
from jax.experimental.pallas import tpu as pltpu
from jax.experimental.pallas import tpu_sc as plsc

_INFO = plsc.get_sparse_core_info()
_NC = _INFO.num_cores
_NS = _INFO.num_subcores
_NW = _NC * _NS
_LANES = _INFO.num_lanes


def _pe_table(max_length, d):
    position = jnp.arange(max_length, dtype=jnp.float32)[:, None]
    col = jnp.arange(d, dtype=jnp.int32)
    div_term = jnp.exp((col // 2 * 2).astype(jnp.float32)
                       * (-math.log(10000.0) / d))
    arg = position * div_term
    return jnp.where((col % 2) == 0, jnp.sin(arg), jnp.cos(arg))


def _build_fused_table(token_embedding, pe, scale):
    V, D = token_embedding.shape
    L = pe.shape[0]

    rows = V * L
    rows_pad = rows + (-rows) % (8 * _NS)

    def body(emb_ref, pe_ref, out_ref):
        pe_block = pe_ref[...]
        for t in range(V):
            out_ref[pl.ds(t * L, L), :] = emb_ref[t] * scale + pe_block
        if rows_pad > rows:
            out_ref[pl.ds(rows, rows_pad - rows), :] = jnp.zeros(
                (rows_pad - rows, D), jnp.float32)

    return pl.pallas_call(
        body,
        out_shape=jax.ShapeDtypeStruct((rows_pad, D), jnp.float32),
    )(token_embedding, pe)


def _sc_lookup(input_tokens, fused, B, L):
    R, D = fused.shape
    seq_per_w = B // _NW
    rows_per_w = seq_per_w * L
    chunk = 128
    n_chunks = rows_per_w // chunk
    n_groups = seq_per_w // _LANES

    mesh = plsc.VectorSubcoreMesh(core_axis_name="c", subcore_axis_name="s")

    @functools.partial(
        pl.kernel,
        out_type=jax.ShapeDtypeStruct((B * L, D), jnp.float32),
        mesh=mesh,
        compiler_params=pltpu.CompilerParams(needs_layout_passes=False),
        scratch_types=[
            pltpu.VMEM((seq_per_w * L,), jnp.int32),
            pltpu.VMEM((n_chunks, chunk), jnp.int32),
            pltpu.VMEM_SHARED((R, D), jnp.float32),
            [pltpu.VMEM((chunk, D), jnp.float32) for _ in range(2)],
            [pltpu.SemaphoreType.DMA for _ in range(2)],
            [pltpu.SemaphoreType.DMA for _ in range(2)],
            pltpu.SemaphoreType.DMA,
        ],
    )
    def sc_kernel(tok_hbm, fused_hbm, out_hbm, tok_v, idx_v, table_sp, bufs,
                  gsems, wsems, fsem):
        n_buf = len(bufs)
        sid = lax.axis_index("s")
        wid = sid * _NC + lax.axis_index("c")
        base_seq = wid * seq_per_w
        rows_per_sub = R // _NS
        fill = pltpu.async_copy(
            fused_hbm.at[pl.ds(sid * rows_per_sub, rows_per_sub)],
            table_sp.at[pl.ds(sid * rows_per_sub, rows_per_sub)], fsem)
        pltpu.sync_copy(
            tok_hbm.at[pl.ds(base_seq * L, seq_per_w * L)], tok_v)

        lane = lax.broadcasted_iota(jnp.int32, (_LANES,), 0)
        zeros = jnp.zeros((_LANES,), jnp.int32)

        bases = [(lane + g * _LANES) * L for g in range(n_groups)]
        p0s = []
        for base16 in bases:
            p0 = plsc.load_gather(tok_v, [base16])
            plsc.store_scatter(idx_v, [base16 >> 7, base16 & 127], p0 * L)
            p0s.append(p0)

        def step(j, carry):
            new = []
            for g in range(n_groups):
                p, nc, ev = carry[g]
                base16 = bases[g]
                c = plsc.load_gather(tok_v, [base16 + j])
                nc = jnp.where((c >= 36) & (c <= 41), 2, nc)
                change_lt12 = (p >= 12) | (nc > 0)
                change = jnp.where(c < 12, change_lt12, p < 12)
                nc = jnp.where(c < 12, nc - 1, nc)
                ev = ev + change.astype(jnp.int32)
                flat = base16 + j
                plsc.store_scatter(idx_v, [flat >> 7, flat & 127], c * L + ev)
                new.append((c, nc, ev))
            return tuple(new)

        lax.fori_loop(1, L, step,
                      tuple((p0, zeros, zeros) for p0 in p0s))

        out_base = wid * rows_per_w

        def start_gather(k, b):
            return pltpu.async_copy(
                table_sp.at[idx_v.at[k]], bufs[b], gsems[b])

        def start_write(k, b):
            return pltpu.async_copy(
                bufs[b], out_hbm.at[pl.ds(out_base + k * chunk, chunk)],
                wsems[b])

        def wait_gather(b):
            pltpu.make_async_copy(
                fused_hbm.at[pl.ds(0, chunk)], bufs[b], gsems[b]).wait()

        def wait_write(b):
            pltpu.make_async_copy(
                fused_hbm.at[pl.ds(0, chunk)], bufs[b], wsems[b]).wait()

        fill.wait()
        plsc.subcore_barrier()

        for b in range(n_buf):
            start_gather(b, b)
        wait_gather(0)
        start_write(0, 0)

        def pipe_body(k, _):
            for b in range(n_buf):

                @pl.when((k % n_buf) == b)
                def _():
                    wait_write(b)
                    start_gather(k, b)

            for b in range(n_buf):

                @pl.when(((k - 1) % n_buf) == b)
                def _():
                    wait_gather(b)
                    start_write(k - 1, b)

            return 0

        lax.fori_loop(n_buf, n_chunks, pipe_body, 0)
        for k in range(n_chunks - 1, n_chunks):
            b = k % n_buf
            wait_gather(b)
            start_write(k, b)
        for k in range(n_chunks - n_buf, n_chunks):
            wait_write(k % n_buf)

    return sc_kernel(input_tokens.reshape(B * L), fused)


def kernel(input_tokens, token_embedding):
    B, L = input_tokens.shape
    V, D = token_embedding.shape
    pe = _pe_table(L, D)
    fused = _build_fused_table(token_embedding, pe, math.sqrt(D))
    out = _sc_lookup(input_tokens, fused, B, L)
    return out.reshape(B, L, D)
